# baseline (speedup 1.0000x reference)
import jax
import jax.numpy as jnp
from jax.experimental import pallas as pl
from jax.experimental.pallas import tpu as pltpu


def _w_expansion(wc, dtype):
    cols = jnp.arange(2 * wc)
    return (jnp.arange(wc)[:, None] == (cols // 2)[None, :]).astype(dtype)


def _conv_row(w_ref, x_ref, b_ref, r):
    acc = jnp.dot(w_ref[...], x_ref[0, :, r, :].astype(jnp.bfloat16),
                  preferred_element_type=jnp.float32)
    return acc + b_ref[...]


def _fpn01_kernel(th0, x0, x1, w0, w1, b0, b1, e1, o0, o1):
    rows0 = []
    for r in range(th0):
        acc = _conv_row(w0, x0, b0, r)
        o0[0, :, r, :] = acc.astype(o0.dtype)
        rows0.append(acc)
    for r in range(2 * th0):
        up = jnp.dot(rows0[r // 2], e1[...], preferred_element_type=jnp.float32)
        acc = _conv_row(w1, x1, b1, r) + up
        o1[0, :, r, :] = acc.astype(o1.dtype)


def _fpn23_kernel(th2, x2, x3, w2, w3, b2, b3, prev, e2, e3, o2, o3):
    rows2 = []
    for r in range(th2):
        up = jnp.dot(prev[0, :, r // 2, :].astype(jnp.float32), e2[...],
                     preferred_element_type=jnp.float32)
        acc = _conv_row(w2, x2, b2, r) + up
        o2[0, :, r, :] = acc.astype(o2.dtype)
        rows2.append(acc)
    for r in range(2 * th2):
        up = jnp.dot(rows2[r // 2], e3[...], preferred_element_type=jnp.float32)
        acc = _conv_row(w3, x3, b3, r) + up
        o3[0, :, r, :] = acc.astype(o3.dtype)


def kernel(feat0, feat1, feat2, feat3, w0, w1, w2, w3, b0, b1, b2, b3):
    B, _, H0, W0 = feat0.shape
    cout = w0.shape[0]
    dtype = feat0.dtype
    c0, c1, c2, c3 = (f.shape[1] for f in (feat0, feat1, feat2, feat3))

    wb = [w.astype(jnp.bfloat16) for w in (w0, w1, w2, w3)]
    bb = [b.reshape(cout, 1).astype(jnp.float32) for b in (b0, b1, b2, b3)]
    e1 = _w_expansion(W0, jnp.float32)
    e2 = _w_expansion(2 * W0, jnp.float32)
    e3 = _w_expansion(4 * W0, jnp.float32)

    def x_spec(c, th, w):
        return pl.BlockSpec((1, c, th, w), lambda b, i: (b, 0, i, 0))

    def const_spec(shape):
        return pl.BlockSpec(shape, lambda b, i: tuple(0 for _ in shape))

    params = pltpu.CompilerParams(
        dimension_semantics=("parallel", "parallel"),
        vmem_limit_bytes=60 * 1024 * 1024)

    th0 = 8
    grid01 = (B, H0 // th0)
    out0, out1 = pl.pallas_call(
        lambda *a: _fpn01_kernel(th0, *a),
        grid=grid01,
        in_specs=[
            x_spec(c0, th0, W0),
            x_spec(c1, 2 * th0, 2 * W0),
            const_spec(wb[0].shape), const_spec(wb[1].shape),
            const_spec(bb[0].shape), const_spec(bb[1].shape),
            const_spec(e1.shape),
        ],
        out_specs=[
            x_spec(cout, th0, W0),
            x_spec(cout, 2 * th0, 2 * W0),
        ],
        out_shape=[
            jax.ShapeDtypeStruct((B, cout, H0, W0), dtype),
            jax.ShapeDtypeStruct((B, cout, 2 * H0, 2 * W0), dtype),
        ],
        compiler_params=params,
    )(feat0, feat1, wb[0], wb[1], bb[0], bb[1], e1)

    th2 = 16
    grid23 = (B, (4 * H0) // th2)
    out2, out3 = pl.pallas_call(
        lambda *a: _fpn23_kernel(th2, *a),
        grid=grid23,
        in_specs=[
            x_spec(c2, th2, 4 * W0),
            x_spec(c3, 2 * th2, 8 * W0),
            const_spec(wb[2].shape), const_spec(wb[3].shape),
            const_spec(bb[2].shape), const_spec(bb[3].shape),
            x_spec(cout, th2 // 2, 2 * W0),
            const_spec(e2.shape), const_spec(e3.shape),
        ],
        out_specs=[
            x_spec(cout, th2, 4 * W0),
            x_spec(cout, 2 * th2, 8 * W0),
        ],
        out_shape=[
            jax.ShapeDtypeStruct((B, cout, 4 * H0, 4 * W0), dtype),
            jax.ShapeDtypeStruct((B, cout, 8 * H0, 8 * W0), dtype),
        ],
        compiler_params=params,
    )(feat2, feat3, wb[2], wb[3], bb[2], bb[3], out1, e2, e3)

    return [out0, out1, out2, out3]


# speedup vs baseline: 3.2868x; 3.2868x over previous
import functools

import jax
import jax.numpy as jnp
from jax.experimental import pallas as pl
from jax.experimental.pallas import tpu as pltpu


def _chunk_expansion(wc, chunk, dtype):
    wf = 2 * wc
    cols = jnp.arange(2 * wf)
    src = (cols % wf) // 2
    e_row = (jnp.arange(wc)[:, None] == src[None, :]).astype(dtype)
    return jnp.kron(jnp.eye(chunk // wc, dtype=dtype), e_row)


def _fpn_kernel(widths, rows, n_strips, *refs):
    xs, ws, bs = refs[0:4], refs[4:8], refs[8:12]
    es, os_ = refs[12:15], refs[15:19]
    xv, ov = refs[19:23], refs[23:27]
    si, so = refs[27:31], refs[31:35]

    b = pl.program_id(0)
    i = pl.program_id(1)
    p = jax.lax.rem(i, 2)

    def issue_inputs(strip, parity):
        for k in range(4):
            w = widths[k]
            ts = rows[k] * w
            if w < 128:
                pltpu.make_async_copy(
                    xs[k].at[b, :, pl.ds(strip * ts, ts)],
                    xv[k].at[parity],
                    si[k].at[parity],
                ).start()
            else:
                for r in range(rows[k]):
                    row = strip * rows[k] + r
                    pltpu.make_async_copy(
                        xs[k].at[b, :, row, :],
                        xv[k].at[parity, :, pl.ds(r * w, w)],
                        si[k].at[parity],
                    ).start()

    @pl.when(i == 0)
    def _():
        issue_inputs(i, p)

    @pl.when(i + 1 < n_strips)
    def _():
        issue_inputs(i + 1, 1 - p)

    for k in range(4):
        w = widths[k]
        ts = rows[k] * w

        @pl.when(i >= 2)
        def _(k=k):
            pltpu.make_async_copy(
                ov[k].at[p], ov[k].at[p], so[k].at[p]).wait()

        pltpu.make_async_copy(xv[k].at[p], xv[k].at[p], si[k].at[p]).wait()

        x = xv[k][p].astype(jnp.bfloat16)
        ov[k][p] = (
            jnp.dot(ws[k][...], x, preferred_element_type=jnp.float32)
            + bs[k][...]
        )
        if k > 0:
            e = es[k - 1][...]
            chunk = e.shape[0]
            for c in range(rows[k - 1] * widths[k - 1] // chunk):
                up = jnp.dot(ov[k - 1][p, :, pl.ds(c * chunk, chunk)],
                             e, preferred_element_type=jnp.float32)
                sl = pl.ds(4 * c * chunk, 4 * chunk)
                ov[k][p, :, sl] = ov[k][p, :, sl] + up

        if w < 128:
            pltpu.make_async_copy(
                ov[k].at[p],
                os_[k].at[b, :, pl.ds(i * ts, ts)],
                so[k].at[p],
            ).start()
        else:
            for r in range(rows[k]):
                row = i * rows[k] + r
                pltpu.make_async_copy(
                    ov[k].at[p, :, pl.ds(r * w, w)],
                    os_[k].at[b, :, row, :],
                    so[k].at[p],
                ).start()

    @pl.when(i == n_strips - 1)
    def _():
        for k in range(4):
            pltpu.make_async_copy(
                ov[k].at[p], ov[k].at[p], so[k].at[p]).wait()
            if n_strips > 1:
                pltpu.make_async_copy(
                    ov[k].at[1 - p], ov[k].at[1 - p], so[k].at[1 - p]).wait()


def kernel(feat0, feat1, feat2, feat3, w0, w1, w2, w3, b0, b1, b2, b3):
    feats = [feat0, feat1, feat2, feat3]
    B, _, H0, W0 = feat0.shape
    cout = w0.shape[0]
    dtype = feat0.dtype
    cins = [f.shape[1] for f in feats]
    widths = tuple(W0 * (1 << k) for k in range(4))

    th0 = 4
    n_strips = H0 // th0
    rows = tuple(th0 * (1 << k) for k in range(4))
    tss = [rows[k] * widths[k] for k in range(4)]

    wb = [w.astype(jnp.bfloat16) for w in (w0, w1, w2, w3)]
    bb = [b.reshape(cout, 1).astype(jnp.float32) for b in (b0, b1, b2, b3)]
    es = []
    for k in (1, 2, 3):
        wc = widths[k - 1]
        chunk = wc * max(1, min(128, tss[k - 1]) // wc)
        es.append(_chunk_expansion(wc, chunk, jnp.float32))

    def const_spec(shape):
        return pl.BlockSpec(shape, lambda b, i: tuple(0 for _ in shape))

    hbm = pl.BlockSpec(memory_space=pl.ANY)

    feats_in = [
        f.reshape(B, cins[k], n_strips * tss[k]) if widths[k] < 128 else f
        for k, f in enumerate(feats)
    ]
    out_shape = [
        jax.ShapeDtypeStruct(
            (B, cout, n_strips * tss[k]) if widths[k] < 128
            else (B, cout, H0 * (1 << k), W0 * (1 << k)), dtype)
        for k in range(4)
    ]

    outs = pl.pallas_call(
        functools.partial(_fpn_kernel, widths, rows, n_strips),
        grid=(B, n_strips),
        in_specs=(
            [hbm] * 4
            + [const_spec(w.shape) for w in wb]
            + [const_spec(b.shape) for b in bb]
            + [const_spec(e.shape) for e in es]
        ),
        out_specs=[hbm] * 4,
        out_shape=out_shape,
        scratch_shapes=(
            [pltpu.VMEM((2, cins[k], tss[k]), dtype) for k in range(4)]
            + [pltpu.VMEM((2, cout, tss[k]), jnp.float32) for k in range(4)]
            + [pltpu.SemaphoreType.DMA((2,)) for _ in range(8)]
        ),
        compiler_params=pltpu.CompilerParams(
            dimension_semantics=("parallel", "arbitrary"),
            vmem_limit_bytes=62 * 1024 * 1024),
    )(*feats_in, *wb, *bb, *es)

    return [
        o.reshape(B, cout, H0 * (1 << k), W0 * (1 << k))
        for k, o in enumerate(outs)
    ]
